# baseline (speedup 1.0000x reference)
import functools

import jax
import jax.numpy as jnp
from jax import lax
from jax.experimental import pallas as pl
from jax.experimental.pallas import tpu as pltpu
from jax.experimental.pallas import tpu_sc as plsc

E = 8
TOP_K = 2
HIDDEN = 1024
FFN = 1408
ALPHA = 0.001
SCALING = 1.0

BLK = 128
NBLK = 40
NPAD = NBLK * BLK

NC = 2
NS = 16
NW = NC * NS
L = 16

T = 2048
TOK_W = T // NW
TOK_S = T // NS
SLOT_W = NPAD // NW
GCH = SLOT_W // 2
CCH = TOK_W // 2


def _gate_kernel(x_ref, gw_ref, pos0_ref, pos1_ref, wt0_ref, wt1_ref,
                 be_ref, laux_ref):
    x = x_ref[...]
    gw = gw_ref[...]
    logits = jax.lax.dot_general(x, gw, (((1,), (1,)), ((), ())),
                                 preferred_element_type=jnp.float32)
    m = jnp.max(logits, axis=-1, keepdims=True)
    ex = jnp.exp(logits - m)
    s = ex / jnp.sum(ex, axis=-1, keepdims=True)
    iota = jax.lax.broadcasted_iota(jnp.int32, s.shape, 1)
    m1 = jnp.max(s, axis=-1, keepdims=True)
    i1 = jnp.min(jnp.where(s == m1, iota, E), axis=-1, keepdims=True)
    s2 = jnp.where(iota == i1, -1.0, s)
    m2 = jnp.max(s2, axis=-1, keepdims=True)
    i2 = jnp.min(jnp.where(s2 == m2, iota, E), axis=-1, keepdims=True)
    denom = m1 + m2 + 1e-20
    wt0_ref[...] = ((m1 / denom) * SCALING)[:, 0]
    wt1_ref[...] = ((m2 / denom) * SCALING)[:, 0]

    oh0 = (iota == i1).astype(jnp.float32)
    oh1 = (iota == i2).astype(jnp.float32)

    r = jax.lax.broadcasted_iota(jnp.int32, (T, T), 0)
    c = jax.lax.broadcasted_iota(jnp.int32, (T, T), 1)
    tril = (r >= c).astype(jnp.float32)
    c0 = jax.lax.dot_general(tril, oh0, (((1,), (0,)), ((), ())),
                             preferred_element_type=jnp.float32)
    c1 = jax.lax.dot_general(tril, oh1, (((1,), (0,)), ((), ())),
                             preferred_element_type=jnp.float32)
    counts0 = jnp.sum(oh0, axis=0)
    counts1 = jnp.sum(oh1, axis=0)
    counts = counts0 + counts1
    rank0 = jnp.sum(c0 * oh0, axis=-1) - 1.0
    rank1 = jnp.sum((c1 + counts0[None, :]) * oh1, axis=-1) - 1.0

    counts_i = counts.astype(jnp.int32)
    padded = ((counts_i + (BLK - 1)) // BLK) * BLK
    er = jax.lax.broadcasted_iota(jnp.int32, (E, E), 0)
    ec = jax.lax.broadcasted_iota(jnp.int32, (E, E), 1)
    offs = jnp.sum(jnp.where(ec < er, padded[None, :], 0), axis=1)
    cumblk = jnp.sum(jnp.where(ec <= er, padded[None, :] // BLK, 0), axis=1)

    pos0_ref[...] = (jnp.sum(oh0 * offs[None, :].astype(jnp.float32), axis=-1)
                     + rank0).astype(jnp.int32)
    pos1_ref[...] = (jnp.sum(oh1 * offs[None, :].astype(jnp.float32), axis=-1)
                     + rank1).astype(jnp.int32)

    bi = jax.lax.broadcasted_iota(jnp.int32, (NBLK, E), 0)
    be = jnp.sum((bi >= cumblk[None, :]).astype(jnp.int32), axis=1)
    be_ref[...] = jnp.minimum(be, E - 1)

    ssum = jnp.sum(s, axis=0)
    laux = jnp.sum(ssum * counts) * (ALPHA * E / (T * TOP_K * T))
    laux_ref[...] = laux.reshape(1, 1)


def _dispatch_kernel(x_hbm, pos0_hbm, pos1_hbm, wt0_hbm, wt1_hbm,
                     xs_hbm, ws_hbm,
                     idx_v, val_v, wv_v, zero_v, zerof_v, sidx_v, rows_v,
                     wsv_v, st_sh, wsum_sh, sem):
    cid = lax.axis_index("c")
    sid = lax.axis_index("s")
    wid = sid * NC + cid

    z_n = NPAD // NS
    for j in range(z_n // L):
        zero_v[pl.ds(j * L, L)] = jnp.zeros((L,), jnp.int32)
        zerof_v[pl.ds(j * L, L)] = jnp.zeros((L,), jnp.float32)
    pltpu.sync_copy(zero_v, st_sh.at[pl.ds(sid * z_n, z_n)])
    pltpu.sync_copy(zerof_v, wsum_sh.at[pl.ds(sid * z_n, z_n)])
    plsc.subcore_barrier()

    base = sid * TOK_S
    for j in range(TOK_S // L):
        val_v[pl.ds(j * L, L)] = (
            lax.broadcasted_iota(jnp.int32, (L,), 0) + (base + j * L))
    pltpu.sync_copy(pos0_hbm.at[pl.ds(base, TOK_S)], idx_v)
    pltpu.sync_copy(val_v, st_sh.at[idx_v], add=True)
    pltpu.sync_copy(wt0_hbm.at[pl.ds(base, TOK_S)], wv_v)
    pltpu.sync_copy(wv_v, wsum_sh.at[idx_v], add=True)
    pltpu.sync_copy(pos1_hbm.at[pl.ds(base, TOK_S)], idx_v)
    pltpu.sync_copy(val_v, st_sh.at[idx_v], add=True)
    pltpu.sync_copy(wt1_hbm.at[pl.ds(base, TOK_S)], wv_v)
    pltpu.sync_copy(wv_v, wsum_sh.at[idx_v], add=True)
    plsc.subcore_barrier()

    sbase = wid * SLOT_W
    for g in range(SLOT_W // GCH):
        gb = sbase + g * GCH
        pltpu.sync_copy(st_sh.at[pl.ds(gb, GCH)], sidx_v)
        pltpu.async_copy(x_hbm.at[sidx_v], rows_v, sem).wait()
        pltpu.sync_copy(rows_v, xs_hbm.at[pl.ds(gb, GCH)])
    pltpu.sync_copy(wsum_sh.at[pl.ds(sbase, SLOT_W)], wsv_v)
    pltpu.sync_copy(wsv_v, ws_hbm.at[pl.ds(sbase, SLOT_W)])


def _expert_kernel(be_ref, xs_ref, w1_ref, w2_ref, ws_ref, ys_ref):
    x = xs_ref[...]
    mid = jnp.dot(x, w1_ref[0], preferred_element_type=jnp.float32)
    g = mid[:, :FFN]
    u = mid[:, FFN:]
    act = g * jax.lax.logistic(g) * u
    y = jnp.dot(act, w2_ref[0], preferred_element_type=jnp.float32)
    ys_ref[...] = y * ws_ref[0, 0][:, None]


def _combine_kernel(ys_hbm, pos0_hbm, pos1_hbm, out_hbm,
                    idx0_v, idx1_v, buf0, buf1, sem):
    cid = lax.axis_index("c")
    sid = lax.axis_index("s")
    wid = sid * NC + cid
    for g in range(TOK_W // CCH):
        base = wid * TOK_W + g * CCH
        pltpu.sync_copy(pos0_hbm.at[pl.ds(base, CCH)], idx0_v)
        pltpu.sync_copy(pos1_hbm.at[pl.ds(base, CCH)], idx1_v)
        pltpu.async_copy(ys_hbm.at[idx0_v], buf0, sem).wait()
        pltpu.async_copy(ys_hbm.at[idx1_v], buf1, sem).wait()

        def row(i, _):
            def col(c, __):
                a = buf0[i, pl.ds(c * L, L)]
                b = buf1[i, pl.ds(c * L, L)]
                buf0[i, pl.ds(c * L, L)] = a + b
                return 0
            return lax.fori_loop(0, HIDDEN // L, col, 0)

        lax.fori_loop(0, CCH, row, 0)
        pltpu.sync_copy(buf0, out_hbm.at[pl.ds(base, CCH)])


def kernel(hidden_states, gate_weight, w1, w2):
    seq, b, h = hidden_states.shape
    x = hidden_states.reshape(T, h)

    pos0, pos1, wt0, wt1, be, laux = pl.pallas_call(
        _gate_kernel,
        out_shape=(
            jax.ShapeDtypeStruct((T,), jnp.int32),
            jax.ShapeDtypeStruct((T,), jnp.int32),
            jax.ShapeDtypeStruct((T,), jnp.float32),
            jax.ShapeDtypeStruct((T,), jnp.float32),
            jax.ShapeDtypeStruct((NBLK,), jnp.int32),
            jax.ShapeDtypeStruct((1, 1), jnp.float32),
        ),
    )(x, gate_weight)

    mesh = plsc.VectorSubcoreMesh(core_axis_name="c", subcore_axis_name="s")
    dispatch = functools.partial(
        pl.kernel,
        out_type=(
            jax.ShapeDtypeStruct((NPAD, HIDDEN), jnp.float32),
            jax.ShapeDtypeStruct((NPAD,), jnp.float32),
        ),
        mesh=mesh,
        scratch_types=[
            pltpu.VMEM((TOK_S,), jnp.int32),
            pltpu.VMEM((TOK_S,), jnp.int32),
            pltpu.VMEM((TOK_S,), jnp.float32),
            pltpu.VMEM((NPAD // NS,), jnp.int32),
            pltpu.VMEM((NPAD // NS,), jnp.float32),
            pltpu.VMEM((GCH,), jnp.int32),
            pltpu.VMEM((GCH, HIDDEN), jnp.float32),
            pltpu.VMEM((SLOT_W,), jnp.float32),
            pltpu.VMEM_SHARED((NPAD,), jnp.int32),
            pltpu.VMEM_SHARED((NPAD,), jnp.float32),
            pltpu.SemaphoreType.DMA,
        ],
    )(_dispatch_kernel)
    xs, ws = dispatch(x, pos0, pos1, wt0, wt1)

    ys = pl.pallas_call(
        _expert_kernel,
        grid_spec=pltpu.PrefetchScalarGridSpec(
            num_scalar_prefetch=1,
            grid=(NBLK,),
            in_specs=[
                pl.BlockSpec((BLK, HIDDEN), lambda i, be_r: (i, 0)),
                pl.BlockSpec((1, HIDDEN, 2 * FFN), lambda i, be_r: (be_r[i], 0, 0)),
                pl.BlockSpec((1, FFN, HIDDEN), lambda i, be_r: (be_r[i], 0, 0)),
                pl.BlockSpec((1, 1, BLK), lambda i, be_r: (i, 0, 0)),
            ],
            out_specs=pl.BlockSpec((BLK, HIDDEN), lambda i, be_r: (i, 0)),
        ),
        out_shape=jax.ShapeDtypeStruct((NPAD, HIDDEN), jnp.float32),
    )(be, xs, w1, w2, ws.reshape(NBLK, 1, BLK))

    combine = functools.partial(
        pl.kernel,
        out_type=jax.ShapeDtypeStruct((T, HIDDEN), jnp.float32),
        mesh=mesh,
        scratch_types=[
            pltpu.VMEM((CCH,), jnp.int32),
            pltpu.VMEM((CCH,), jnp.int32),
            pltpu.VMEM((CCH, HIDDEN), jnp.float32),
            pltpu.VMEM((CCH, HIDDEN), jnp.float32),
            pltpu.SemaphoreType.DMA,
        ],
    )(_combine_kernel)
    out = combine(ys, pos0, pos1)

    out = out.reshape(seq, b, h)
    return out, laux[0, 0]


# speedup vs baseline: 1.0182x; 1.0182x over previous
import functools

import jax
import jax.numpy as jnp
from jax import lax
from jax.experimental import pallas as pl
from jax.experimental.pallas import tpu as pltpu
from jax.experimental.pallas import tpu_sc as plsc

E = 8
TOP_K = 2
HIDDEN = 1024
FFN = 1408
ALPHA = 0.001
SCALING = 1.0

BLK = 128
NBLK = 40
NPAD = NBLK * BLK

NC = 2
NS = 16
NW = NC * NS
L = 16

T = 2048
TOK_W = T // NW
TOK_S = T // NS
SLOT_W = NPAD // NW
GCH = 32
NCHD = SLOT_W // GCH
CCH = 16
NCHC = TOK_W // CCH


def _gate_kernel(x_ref, gw_ref, pos0_ref, pos1_ref, wt0_ref, wt1_ref,
                 be_ref, laux_ref):
    x = x_ref[...]
    gw = gw_ref[...]
    logits = jax.lax.dot_general(x, gw, (((1,), (1,)), ((), ())),
                                 preferred_element_type=jnp.float32)
    m = jnp.max(logits, axis=-1, keepdims=True)
    ex = jnp.exp(logits - m)
    s = ex / jnp.sum(ex, axis=-1, keepdims=True)
    iota = jax.lax.broadcasted_iota(jnp.int32, s.shape, 1)
    m1 = jnp.max(s, axis=-1, keepdims=True)
    i1 = jnp.min(jnp.where(s == m1, iota, E), axis=-1, keepdims=True)
    s2 = jnp.where(iota == i1, -1.0, s)
    m2 = jnp.max(s2, axis=-1, keepdims=True)
    i2 = jnp.min(jnp.where(s2 == m2, iota, E), axis=-1, keepdims=True)
    denom = m1 + m2 + 1e-20
    wt0_ref[...] = ((m1 / denom) * SCALING)[:, 0]
    wt1_ref[...] = ((m2 / denom) * SCALING)[:, 0]

    oh0 = (iota == i1).astype(jnp.float32)
    oh1 = (iota == i2).astype(jnp.float32)

    r = jax.lax.broadcasted_iota(jnp.int32, (T, T), 0)
    c = jax.lax.broadcasted_iota(jnp.int32, (T, T), 1)
    tril = (r >= c).astype(jnp.float32)
    c0 = jax.lax.dot_general(tril, oh0, (((1,), (0,)), ((), ())),
                             preferred_element_type=jnp.float32)
    c1 = jax.lax.dot_general(tril, oh1, (((1,), (0,)), ((), ())),
                             preferred_element_type=jnp.float32)
    counts0 = jnp.sum(oh0, axis=0)
    counts1 = jnp.sum(oh1, axis=0)
    counts = counts0 + counts1
    rank0 = jnp.sum(c0 * oh0, axis=-1) - 1.0
    rank1 = jnp.sum((c1 + counts0[None, :]) * oh1, axis=-1) - 1.0

    counts_i = counts.astype(jnp.int32)
    padded = ((counts_i + (BLK - 1)) // BLK) * BLK
    er = jax.lax.broadcasted_iota(jnp.int32, (E, E), 0)
    ec = jax.lax.broadcasted_iota(jnp.int32, (E, E), 1)
    offs = jnp.sum(jnp.where(ec < er, padded[None, :], 0), axis=1)
    cumblk = jnp.sum(jnp.where(ec <= er, padded[None, :] // BLK, 0), axis=1)

    pos0_ref[...] = (jnp.sum(oh0 * offs[None, :].astype(jnp.float32), axis=-1)
                     + rank0).astype(jnp.int32)
    pos1_ref[...] = (jnp.sum(oh1 * offs[None, :].astype(jnp.float32), axis=-1)
                     + rank1).astype(jnp.int32)

    bi = jax.lax.broadcasted_iota(jnp.int32, (NBLK, E), 0)
    be = jnp.sum((bi >= cumblk[None, :]).astype(jnp.int32), axis=1)
    be_ref[...] = jnp.minimum(be, E - 1)

    ssum = jnp.sum(s, axis=0)
    laux = jnp.sum(ssum * counts) * (ALPHA * E / (T * TOP_K * T))
    laux_ref[...] = laux.reshape(1, 1)


def _dispatch_kernel(x_hbm, pos0_hbm, pos1_hbm, wt0_hbm, wt1_hbm,
                     xs_hbm, ws_hbm,
                     idx_v, val_v, wv_v, zero_v, zerof_v,
                     sidx0_v, sidx1_v, rows0_v, rows1_v,
                     wsv_v, st_sh, wsum_sh, sem0, sem1):
    cid = lax.axis_index("c")
    sid = lax.axis_index("s")
    wid = sid * NC + cid

    z_n = NPAD // NS
    for j in range(z_n // L):
        zero_v[pl.ds(j * L, L)] = jnp.zeros((L,), jnp.int32)
        zerof_v[pl.ds(j * L, L)] = jnp.zeros((L,), jnp.float32)
    pltpu.sync_copy(zero_v, st_sh.at[pl.ds(sid * z_n, z_n)])
    pltpu.sync_copy(zerof_v, wsum_sh.at[pl.ds(sid * z_n, z_n)])
    plsc.subcore_barrier()

    base = sid * TOK_S
    for j in range(TOK_S // L):
        val_v[pl.ds(j * L, L)] = (
            lax.broadcasted_iota(jnp.int32, (L,), 0) + (base + j * L))
    pltpu.sync_copy(pos0_hbm.at[pl.ds(base, TOK_S)], idx_v)
    pltpu.sync_copy(val_v, st_sh.at[idx_v], add=True)
    pltpu.sync_copy(wt0_hbm.at[pl.ds(base, TOK_S)], wv_v)
    pltpu.sync_copy(wv_v, wsum_sh.at[idx_v], add=True)
    pltpu.sync_copy(pos1_hbm.at[pl.ds(base, TOK_S)], idx_v)
    pltpu.sync_copy(val_v, st_sh.at[idx_v], add=True)
    pltpu.sync_copy(wt1_hbm.at[pl.ds(base, TOK_S)], wv_v)
    pltpu.sync_copy(wv_v, wsum_sh.at[idx_v], add=True)
    plsc.subcore_barrier()

    sbase = wid * SLOT_W
    idxs = (sidx0_v, sidx1_v)
    bufs = (rows0_v, rows1_v)
    sems = (sem0, sem1)
    cps = [None, None]
    for g in range(NCHD):
        b = g & 1
        if g >= 2:
            cps[b].wait()
            pltpu.sync_copy(bufs[b], xs_hbm.at[pl.ds(sbase + (g - 2) * GCH, GCH)])
        pltpu.sync_copy(st_sh.at[pl.ds(sbase + g * GCH, GCH)], idxs[b])
        cps[b] = pltpu.async_copy(x_hbm.at[idxs[b]], bufs[b], sems[b])
    for g in range(NCHD - 2, NCHD):
        b = g & 1
        cps[b].wait()
        pltpu.sync_copy(bufs[b], xs_hbm.at[pl.ds(sbase + g * GCH, GCH)])
    pltpu.sync_copy(wsum_sh.at[pl.ds(sbase, SLOT_W)], wsv_v)
    pltpu.sync_copy(wsv_v, ws_hbm.at[pl.ds(sbase, SLOT_W)])


def _expert_kernel(be_ref, xs_ref, w1_ref, w2_ref, ws_ref, ys_ref):
    x = xs_ref[...]
    mid = jnp.dot(x, w1_ref[0], preferred_element_type=jnp.float32)
    g = mid[:, :FFN]
    u = mid[:, FFN:]
    act = g * jax.lax.logistic(g) * u
    y = jnp.dot(act, w2_ref[0], preferred_element_type=jnp.float32)
    ys_ref[...] = y * ws_ref[0, 0][:, None]


def _combine_kernel(ys_hbm, pos0_hbm, pos1_hbm, out_hbm,
                    idx0a, idx1a, idx0b, idx1b,
                    buf0a, buf1a, buf0b, buf1b, sema, semb):
    cid = lax.axis_index("c")
    sid = lax.axis_index("s")
    wid = sid * NC + cid
    wbase = wid * TOK_W
    idx0s = (idx0a, idx0b)
    idx1s = (idx1a, idx1b)
    buf0s = (buf0a, buf0b)
    buf1s = (buf1a, buf1b)
    sems = (sema, semb)
    cp0s = [None, None]
    cp1s = [None, None]

    def process(b, g):
        cp0s[b].wait()
        cp1s[b].wait()
        b0 = buf0s[b]
        b1 = buf1s[b]

        def row(i, _):
            def col(c, __):
                b0[i, pl.ds(c * L, L)] = (b0[i, pl.ds(c * L, L)]
                                          + b1[i, pl.ds(c * L, L)])
                return 0
            return lax.fori_loop(0, HIDDEN // L, col, 0)

        lax.fori_loop(0, CCH, row, 0)
        pltpu.sync_copy(b0, out_hbm.at[pl.ds(wbase + g * CCH, CCH)])

    for g in range(NCHC):
        b = g & 1
        if g >= 2:
            process(b, g - 2)
        base = wbase + g * CCH
        pltpu.sync_copy(pos0_hbm.at[pl.ds(base, CCH)], idx0s[b])
        pltpu.sync_copy(pos1_hbm.at[pl.ds(base, CCH)], idx1s[b])
        cp0s[b] = pltpu.async_copy(ys_hbm.at[idx0s[b]], buf0s[b], sems[b])
        cp1s[b] = pltpu.async_copy(ys_hbm.at[idx1s[b]], buf1s[b], sems[b])
    for g in range(NCHC - 2, NCHC):
        process(g & 1, g)


def kernel(hidden_states, gate_weight, w1, w2):
    seq, b, h = hidden_states.shape
    x = hidden_states.reshape(T, h)

    pos0, pos1, wt0, wt1, be, laux = pl.pallas_call(
        _gate_kernel,
        out_shape=(
            jax.ShapeDtypeStruct((T,), jnp.int32),
            jax.ShapeDtypeStruct((T,), jnp.int32),
            jax.ShapeDtypeStruct((T,), jnp.float32),
            jax.ShapeDtypeStruct((T,), jnp.float32),
            jax.ShapeDtypeStruct((NBLK,), jnp.int32),
            jax.ShapeDtypeStruct((1, 1), jnp.float32),
        ),
    )(x, gate_weight)

    mesh = plsc.VectorSubcoreMesh(core_axis_name="c", subcore_axis_name="s")
    dispatch = functools.partial(
        pl.kernel,
        out_type=(
            jax.ShapeDtypeStruct((NPAD, HIDDEN), jnp.float32),
            jax.ShapeDtypeStruct((NPAD,), jnp.float32),
        ),
        mesh=mesh,
        scratch_types=[
            pltpu.VMEM((TOK_S,), jnp.int32),
            pltpu.VMEM((TOK_S,), jnp.int32),
            pltpu.VMEM((TOK_S,), jnp.float32),
            pltpu.VMEM((NPAD // NS,), jnp.int32),
            pltpu.VMEM((NPAD // NS,), jnp.float32),
            pltpu.VMEM((GCH,), jnp.int32),
            pltpu.VMEM((GCH,), jnp.int32),
            pltpu.VMEM((GCH, HIDDEN), jnp.float32),
            pltpu.VMEM((GCH, HIDDEN), jnp.float32),
            pltpu.VMEM((SLOT_W,), jnp.float32),
            pltpu.VMEM_SHARED((NPAD,), jnp.int32),
            pltpu.VMEM_SHARED((NPAD,), jnp.float32),
            pltpu.SemaphoreType.DMA,
            pltpu.SemaphoreType.DMA,
        ],
    )(_dispatch_kernel)
    xs, ws = dispatch(x, pos0, pos1, wt0, wt1)

    ys = pl.pallas_call(
        _expert_kernel,
        grid_spec=pltpu.PrefetchScalarGridSpec(
            num_scalar_prefetch=1,
            grid=(NBLK,),
            in_specs=[
                pl.BlockSpec((BLK, HIDDEN), lambda i, be_r: (i, 0)),
                pl.BlockSpec((1, HIDDEN, 2 * FFN), lambda i, be_r: (be_r[i], 0, 0)),
                pl.BlockSpec((1, FFN, HIDDEN), lambda i, be_r: (be_r[i], 0, 0)),
                pl.BlockSpec((1, 1, BLK), lambda i, be_r: (i, 0, 0)),
            ],
            out_specs=pl.BlockSpec((BLK, HIDDEN), lambda i, be_r: (i, 0)),
        ),
        out_shape=jax.ShapeDtypeStruct((NPAD, HIDDEN), jnp.float32),
    )(be, xs, w1, w2, ws.reshape(NBLK, 1, BLK))

    combine = functools.partial(
        pl.kernel,
        out_type=jax.ShapeDtypeStruct((T, HIDDEN), jnp.float32),
        mesh=mesh,
        scratch_types=[
            pltpu.VMEM((CCH,), jnp.int32),
            pltpu.VMEM((CCH,), jnp.int32),
            pltpu.VMEM((CCH,), jnp.int32),
            pltpu.VMEM((CCH,), jnp.int32),
            pltpu.VMEM((CCH, HIDDEN), jnp.float32),
            pltpu.VMEM((CCH, HIDDEN), jnp.float32),
            pltpu.VMEM((CCH, HIDDEN), jnp.float32),
            pltpu.VMEM((CCH, HIDDEN), jnp.float32),
            pltpu.SemaphoreType.DMA,
            pltpu.SemaphoreType.DMA,
        ],
    )(_combine_kernel)
    out = combine(ys, pos0, pos1)

    out = out.reshape(seq, b, h)
    return out, laux[0, 0]


# speedup vs baseline: 1.0195x; 1.0013x over previous
import functools

import jax
import jax.numpy as jnp
from jax import lax
from jax.experimental import pallas as pl
from jax.experimental.pallas import tpu as pltpu
from jax.experimental.pallas import tpu_sc as plsc

E = 8
TOP_K = 2
HIDDEN = 1024
FFN = 1408
ALPHA = 0.001
SCALING = 1.0

BLK = 128
NBLK = 40
NPAD = NBLK * BLK

NC = 2
NS = 16
NW = NC * NS
L = 16

T = 2048
TOK_W = T // NW
TOK_S = T // NS
SLOT_W = NPAD // NW
GCH = 32
NCHD = SLOT_W // GCH
CCH = 16
NCHC = TOK_W // CCH


def _gate_kernel(x_ref, gw_ref, pos0_ref, pos1_ref, wt0_ref, wt1_ref,
                 be_ref, laux_ref):
    x = x_ref[...]
    gw = gw_ref[...]
    logits = jax.lax.dot_general(x, gw, (((1,), (1,)), ((), ())),
                                 preferred_element_type=jnp.float32)
    m = jnp.max(logits, axis=-1, keepdims=True)
    ex = jnp.exp(logits - m)
    s = ex / jnp.sum(ex, axis=-1, keepdims=True)
    iota = jax.lax.broadcasted_iota(jnp.int32, s.shape, 1)
    m1 = jnp.max(s, axis=-1, keepdims=True)
    i1 = jnp.min(jnp.where(s == m1, iota, E), axis=-1, keepdims=True)
    s2 = jnp.where(iota == i1, -1.0, s)
    m2 = jnp.max(s2, axis=-1, keepdims=True)
    i2 = jnp.min(jnp.where(s2 == m2, iota, E), axis=-1, keepdims=True)
    denom = m1 + m2 + 1e-20
    wt0_ref[...] = ((m1 / denom) * SCALING)[:, 0]
    wt1_ref[...] = ((m2 / denom) * SCALING)[:, 0]

    oh0 = (iota == i1).astype(jnp.float32)
    oh1 = (iota == i2).astype(jnp.float32)

    r = jax.lax.broadcasted_iota(jnp.int32, (T, T), 0)
    c = jax.lax.broadcasted_iota(jnp.int32, (T, T), 1)
    tril = (r >= c).astype(jnp.float32)
    c0 = jax.lax.dot_general(tril, oh0, (((1,), (0,)), ((), ())),
                             preferred_element_type=jnp.float32)
    c1 = jax.lax.dot_general(tril, oh1, (((1,), (0,)), ((), ())),
                             preferred_element_type=jnp.float32)
    counts0 = jnp.sum(oh0, axis=0)
    counts1 = jnp.sum(oh1, axis=0)
    counts = counts0 + counts1
    rank0 = jnp.sum(c0 * oh0, axis=-1) - 1.0
    rank1 = jnp.sum((c1 + counts0[None, :]) * oh1, axis=-1) - 1.0

    counts_i = counts.astype(jnp.int32)
    padded = ((counts_i + (BLK - 1)) // BLK) * BLK
    er = jax.lax.broadcasted_iota(jnp.int32, (E, E), 0)
    ec = jax.lax.broadcasted_iota(jnp.int32, (E, E), 1)
    offs = jnp.sum(jnp.where(ec < er, padded[None, :], 0), axis=1)
    cumblk = jnp.sum(jnp.where(ec <= er, padded[None, :] // BLK, 0), axis=1)

    pos0_ref[...] = (jnp.sum(oh0 * offs[None, :].astype(jnp.float32), axis=-1)
                     + rank0).astype(jnp.int32)
    pos1_ref[...] = (jnp.sum(oh1 * offs[None, :].astype(jnp.float32), axis=-1)
                     + rank1).astype(jnp.int32)

    bi = jax.lax.broadcasted_iota(jnp.int32, (NBLK, E), 0)
    be = jnp.sum((bi >= cumblk[None, :]).astype(jnp.int32), axis=1)
    be_ref[...] = jnp.minimum(be, E - 1)

    ssum = jnp.sum(s, axis=0)
    laux = jnp.sum(ssum * counts) * (ALPHA * E / (T * TOP_K * T))
    laux_ref[...] = laux.reshape(1, 1)


def _dispatch_kernel(x_hbm, pos0_hbm, pos1_hbm, wt0_hbm, wt1_hbm,
                     xs_hbm, ws_hbm,
                     idx_v, val_v, wv_v, zero_v, zerof_v,
                     sidx0_v, sidx1_v, rows0_v, rows1_v,
                     wsv_v, st_sh, wsum_sh, sem0, sem1):
    cid = lax.axis_index("c")
    sid = lax.axis_index("s")
    wid = sid * NC + cid

    with jax.named_scope("disp_zero"):
        z_n = NPAD // NS
        for j in range(z_n // L):
            zero_v[pl.ds(j * L, L)] = jnp.zeros((L,), jnp.int32)
            zerof_v[pl.ds(j * L, L)] = jnp.zeros((L,), jnp.float32)
        pltpu.sync_copy(zero_v, st_sh.at[pl.ds(sid * z_n, z_n)])
        pltpu.sync_copy(zerof_v, wsum_sh.at[pl.ds(sid * z_n, z_n)])
        plsc.subcore_barrier()

    with jax.named_scope("disp_scatter"):
        base = sid * TOK_S
        for j in range(TOK_S // L):
            val_v[pl.ds(j * L, L)] = (
                lax.broadcasted_iota(jnp.int32, (L,), 0) + (base + j * L))
        pltpu.sync_copy(pos0_hbm.at[pl.ds(base, TOK_S)], idx_v)
        pltpu.sync_copy(val_v, st_sh.at[idx_v], add=True)
        pltpu.sync_copy(wt0_hbm.at[pl.ds(base, TOK_S)], wv_v)
        pltpu.sync_copy(wv_v, wsum_sh.at[idx_v], add=True)
        pltpu.sync_copy(pos1_hbm.at[pl.ds(base, TOK_S)], idx_v)
        pltpu.sync_copy(val_v, st_sh.at[idx_v], add=True)
        pltpu.sync_copy(wt1_hbm.at[pl.ds(base, TOK_S)], wv_v)
        pltpu.sync_copy(wv_v, wsum_sh.at[idx_v], add=True)
        plsc.subcore_barrier()

    with jax.named_scope("disp_gather"):
        sbase = wid * SLOT_W
        idxs = (sidx0_v, sidx1_v)
        bufs = (rows0_v, rows1_v)
        sems = (sem0, sem1)
        cps = [None, None]
        for g in range(NCHD):
            b = g & 1
            if g >= 2:
                cps[b].wait()
                pltpu.sync_copy(bufs[b],
                                xs_hbm.at[pl.ds(sbase + (g - 2) * GCH, GCH)])
            pltpu.sync_copy(st_sh.at[pl.ds(sbase + g * GCH, GCH)], idxs[b])
            cps[b] = pltpu.async_copy(x_hbm.at[idxs[b]], bufs[b], sems[b])
        for g in range(NCHD - 2, NCHD):
            b = g & 1
            cps[b].wait()
            pltpu.sync_copy(bufs[b], xs_hbm.at[pl.ds(sbase + g * GCH, GCH)])
        pltpu.sync_copy(wsum_sh.at[pl.ds(sbase, SLOT_W)], wsv_v)
        pltpu.sync_copy(wsv_v, ws_hbm.at[pl.ds(sbase, SLOT_W)])


def _expert_kernel(be_ref, xs_ref, w1_ref, w2_ref, ws_ref, ys_ref):
    x = xs_ref[...]
    mid = jnp.dot(x, w1_ref[0], preferred_element_type=jnp.float32)
    g = mid[:, :FFN]
    u = mid[:, FFN:]
    act = g * jax.lax.logistic(g) * u
    y = jnp.dot(act, w2_ref[0], preferred_element_type=jnp.float32)
    ys_ref[...] = y * ws_ref[0, 0][:, None]


def _combine_kernel(ys_hbm, pos0_hbm, pos1_hbm, out_hbm,
                    idx0a, idx1a, idx0b, idx1b,
                    buf0a, buf1a, buf0b, buf1b, sema, semb):
    cid = lax.axis_index("c")
    sid = lax.axis_index("s")
    wid = sid * NC + cid
    wbase = wid * TOK_W
    idx0s = (idx0a, idx0b)
    idx1s = (idx1a, idx1b)
    buf0s = (buf0a, buf0b)
    buf1s = (buf1a, buf1b)
    sems = (sema, semb)
    cp0s = [None, None]
    cp1s = [None, None]

    def process(b, g):
        cp0s[b].wait()
        cp1s[b].wait()
        b0 = buf0s[b]
        b1 = buf1s[b]

        def row(i, _):
            def col(c, __):
                b0[i, pl.ds(c * L, L)] = (b0[i, pl.ds(c * L, L)]
                                          + b1[i, pl.ds(c * L, L)])
                return 0
            return lax.fori_loop(0, HIDDEN // L, col, 0)

        lax.fori_loop(0, CCH, row, 0)
        pltpu.sync_copy(b0, out_hbm.at[pl.ds(wbase + g * CCH, CCH)])

    for g in range(NCHC):
        b = g & 1
        if g >= 2:
            process(b, g - 2)
        base = wbase + g * CCH
        pltpu.sync_copy(pos0_hbm.at[pl.ds(base, CCH)], idx0s[b])
        pltpu.sync_copy(pos1_hbm.at[pl.ds(base, CCH)], idx1s[b])
        cp0s[b] = pltpu.async_copy(ys_hbm.at[idx0s[b]], buf0s[b], sems[b])
        cp1s[b] = pltpu.async_copy(ys_hbm.at[idx1s[b]], buf1s[b], sems[b])
    for g in range(NCHC - 2, NCHC):
        process(g & 1, g)


def kernel(hidden_states, gate_weight, w1, w2):
    seq, b, h = hidden_states.shape
    x = hidden_states.reshape(T, h)

    pos0, pos1, wt0, wt1, be, laux = pl.pallas_call(
        _gate_kernel,
        out_shape=(
            jax.ShapeDtypeStruct((T,), jnp.int32),
            jax.ShapeDtypeStruct((T,), jnp.int32),
            jax.ShapeDtypeStruct((T,), jnp.float32),
            jax.ShapeDtypeStruct((T,), jnp.float32),
            jax.ShapeDtypeStruct((NBLK,), jnp.int32),
            jax.ShapeDtypeStruct((1, 1), jnp.float32),
        ),
    )(x, gate_weight)

    mesh = plsc.VectorSubcoreMesh(core_axis_name="c", subcore_axis_name="s")
    dispatch = functools.partial(
        pl.kernel,
        out_type=(
            jax.ShapeDtypeStruct((NPAD, HIDDEN), jnp.float32),
            jax.ShapeDtypeStruct((NPAD,), jnp.float32),
        ),
        mesh=mesh,
        scratch_types=[
            pltpu.VMEM((TOK_S,), jnp.int32),
            pltpu.VMEM((TOK_S,), jnp.int32),
            pltpu.VMEM((TOK_S,), jnp.float32),
            pltpu.VMEM((NPAD // NS,), jnp.int32),
            pltpu.VMEM((NPAD // NS,), jnp.float32),
            pltpu.VMEM((GCH,), jnp.int32),
            pltpu.VMEM((GCH,), jnp.int32),
            pltpu.VMEM((GCH, HIDDEN), jnp.float32),
            pltpu.VMEM((GCH, HIDDEN), jnp.float32),
            pltpu.VMEM((SLOT_W,), jnp.float32),
            pltpu.VMEM_SHARED((NPAD,), jnp.int32),
            pltpu.VMEM_SHARED((NPAD,), jnp.float32),
            pltpu.SemaphoreType.DMA,
            pltpu.SemaphoreType.DMA,
        ],
    )(_dispatch_kernel)
    xs, ws = dispatch(x, pos0, pos1, wt0, wt1)

    ys = pl.pallas_call(
        _expert_kernel,
        grid_spec=pltpu.PrefetchScalarGridSpec(
            num_scalar_prefetch=1,
            grid=(NBLK,),
            in_specs=[
                pl.BlockSpec((BLK, HIDDEN), lambda i, be_r: (i, 0)),
                pl.BlockSpec((1, HIDDEN, 2 * FFN), lambda i, be_r: (be_r[i], 0, 0)),
                pl.BlockSpec((1, FFN, HIDDEN), lambda i, be_r: (be_r[i], 0, 0)),
                pl.BlockSpec((1, 1, BLK), lambda i, be_r: (i, 0, 0)),
            ],
            out_specs=pl.BlockSpec((BLK, HIDDEN), lambda i, be_r: (i, 0)),
        ),
        out_shape=jax.ShapeDtypeStruct((NPAD, HIDDEN), jnp.float32),
    )(be, xs, w1, w2, ws.reshape(NBLK, 1, BLK))

    combine = functools.partial(
        pl.kernel,
        out_type=jax.ShapeDtypeStruct((T, HIDDEN), jnp.float32),
        mesh=mesh,
        scratch_types=[
            pltpu.VMEM((CCH,), jnp.int32),
            pltpu.VMEM((CCH,), jnp.int32),
            pltpu.VMEM((CCH,), jnp.int32),
            pltpu.VMEM((CCH,), jnp.int32),
            pltpu.VMEM((CCH, HIDDEN), jnp.float32),
            pltpu.VMEM((CCH, HIDDEN), jnp.float32),
            pltpu.VMEM((CCH, HIDDEN), jnp.float32),
            pltpu.VMEM((CCH, HIDDEN), jnp.float32),
            pltpu.SemaphoreType.DMA,
            pltpu.SemaphoreType.DMA,
        ],
    )(_combine_kernel)
    out = combine(ys, pos0, pos1)

    out = out.reshape(seq, b, h)
    return out, laux[0, 0]


# speedup vs baseline: 1.0311x; 1.0114x over previous
import functools

import jax
import jax.numpy as jnp
from jax import lax
from jax.experimental import pallas as pl
from jax.experimental.pallas import tpu as pltpu
from jax.experimental.pallas import tpu_sc as plsc

E = 8
TOP_K = 2
HIDDEN = 1024
FFN = 1408
ALPHA = 0.001
SCALING = 1.0

BLK = 128
NBLK = 40
NPAD = NBLK * BLK

NC = 2
NS = 16
NW = NC * NS
L = 16

T = 2048
TOK_W = T // NW
TOK_S = T // NS
SLOT_W = NPAD // NW
GCH = 32
NCHD = SLOT_W // GCH
CCH = 16
NCHC = TOK_W // CCH


def _gate_kernel(x_ref, gw_ref, pos0_ref, pos1_ref, wt0_ref, wt1_ref,
                 be_ref, laux_ref):
    x = x_ref[...]
    gw = gw_ref[...]
    logits = jax.lax.dot_general(x, gw, (((1,), (1,)), ((), ())),
                                 preferred_element_type=jnp.float32)
    m = jnp.max(logits, axis=-1, keepdims=True)
    ex = jnp.exp(logits - m)
    s = ex / jnp.sum(ex, axis=-1, keepdims=True)
    iota = jax.lax.broadcasted_iota(jnp.int32, s.shape, 1)
    m1 = jnp.max(s, axis=-1, keepdims=True)
    i1 = jnp.min(jnp.where(s == m1, iota, E), axis=-1, keepdims=True)
    s2 = jnp.where(iota == i1, -1.0, s)
    m2 = jnp.max(s2, axis=-1, keepdims=True)
    i2 = jnp.min(jnp.where(s2 == m2, iota, E), axis=-1, keepdims=True)
    denom = m1 + m2 + 1e-20
    wt0_ref[...] = ((m1 / denom) * SCALING)[:, 0]
    wt1_ref[...] = ((m2 / denom) * SCALING)[:, 0]

    oh0 = (iota == i1).astype(jnp.float32)
    oh1 = (iota == i2).astype(jnp.float32)

    CH = 256
    r = jax.lax.broadcasted_iota(jnp.int32, (CH, CH), 0)
    c = jax.lax.broadcasted_iota(jnp.int32, (CH, CH), 1)
    tril = (r >= c).astype(jnp.float32)
    ohb = jnp.concatenate([oh0, oh1], axis=1)
    parts = []
    carry = jnp.zeros((1, 2 * E), jnp.float32)
    for j in range(T // CH):
        blk = ohb[j * CH:(j + 1) * CH, :]
        loc = jax.lax.dot_general(tril, blk, (((1,), (0,)), ((), ())),
                                  preferred_element_type=jnp.float32)
        parts.append(loc + carry)
        carry = carry + loc[CH - 1:CH, :]
    cb = jnp.concatenate(parts, axis=0)
    c0 = cb[:, :E]
    c1 = cb[:, E:]
    counts0 = jnp.sum(oh0, axis=0)
    counts1 = jnp.sum(oh1, axis=0)
    counts = counts0 + counts1
    rank0 = jnp.sum(c0 * oh0, axis=-1) - 1.0
    rank1 = jnp.sum((c1 + counts0[None, :]) * oh1, axis=-1) - 1.0

    counts_i = counts.astype(jnp.int32)
    padded = ((counts_i + (BLK - 1)) // BLK) * BLK
    er = jax.lax.broadcasted_iota(jnp.int32, (E, E), 0)
    ec = jax.lax.broadcasted_iota(jnp.int32, (E, E), 1)
    offs = jnp.sum(jnp.where(ec < er, padded[None, :], 0), axis=1)
    cumblk = jnp.sum(jnp.where(ec <= er, padded[None, :] // BLK, 0), axis=1)

    pos0_ref[...] = (jnp.sum(oh0 * offs[None, :].astype(jnp.float32), axis=-1)
                     + rank0).astype(jnp.int32)
    pos1_ref[...] = (jnp.sum(oh1 * offs[None, :].astype(jnp.float32), axis=-1)
                     + rank1).astype(jnp.int32)

    bi = jax.lax.broadcasted_iota(jnp.int32, (NBLK, E), 0)
    be = jnp.sum((bi >= cumblk[None, :]).astype(jnp.int32), axis=1)
    be_ref[...] = jnp.minimum(be, E - 1)

    ssum = jnp.sum(s, axis=0)
    laux = jnp.sum(ssum * counts) * (ALPHA * E / (T * TOP_K * T))
    laux_ref[...] = laux.reshape(1, 1)


def _dispatch_kernel(x_hbm, pos0_hbm, pos1_hbm, wt0_hbm, wt1_hbm,
                     xs_hbm, ws_hbm,
                     idx_v, val_v, wv_v, zero_v, zerof_v,
                     sidx0_v, sidx1_v, rows0_v, rows1_v,
                     wsv_v, st_sh, wsum_sh, sem0, sem1):
    cid = lax.axis_index("c")
    sid = lax.axis_index("s")
    wid = sid * NC + cid

    with jax.named_scope("disp_zero"):
        z_n = NPAD // NS
        for j in range(z_n // L):
            zero_v[pl.ds(j * L, L)] = jnp.zeros((L,), jnp.int32)
            zerof_v[pl.ds(j * L, L)] = jnp.zeros((L,), jnp.float32)
        pltpu.sync_copy(zero_v, st_sh.at[pl.ds(sid * z_n, z_n)])
        pltpu.sync_copy(zerof_v, wsum_sh.at[pl.ds(sid * z_n, z_n)])
        plsc.subcore_barrier()

    with jax.named_scope("disp_scatter"):
        base = sid * TOK_S
        if True:
            for j in range(TOK_S // L):
                val_v[pl.ds(j * L, L)] = (
                    lax.broadcasted_iota(jnp.int32, (L,), 0) + (base + j * L))
            pltpu.sync_copy(pos0_hbm.at[pl.ds(base, TOK_S)], idx_v)
            pltpu.sync_copy(val_v, st_sh.at[idx_v], add=True)
            pltpu.sync_copy(wt0_hbm.at[pl.ds(base, TOK_S)], wv_v)
            pltpu.sync_copy(wv_v, wsum_sh.at[idx_v], add=True)
            pltpu.sync_copy(pos1_hbm.at[pl.ds(base, TOK_S)], idx_v)
            pltpu.sync_copy(val_v, st_sh.at[idx_v], add=True)
            pltpu.sync_copy(wt1_hbm.at[pl.ds(base, TOK_S)], wv_v)
            pltpu.sync_copy(wv_v, wsum_sh.at[idx_v], add=True)
        plsc.subcore_barrier()

    with jax.named_scope("disp_gather"):
        sbase = wid * SLOT_W
        idxs = (sidx0_v, sidx1_v)
        bufs = (rows0_v, rows1_v)
        sems = (sem0, sem1)
        cps = [None, None]
        for g in range(NCHD):
            b = g & 1
            if g >= 2:
                cps[b].wait()
                pltpu.sync_copy(bufs[b],
                                xs_hbm.at[pl.ds(sbase + (g - 2) * GCH, GCH)])
            pltpu.sync_copy(st_sh.at[pl.ds(sbase + g * GCH, GCH)], idxs[b])
            cps[b] = pltpu.async_copy(x_hbm.at[idxs[b]], bufs[b], sems[b])
        for g in range(NCHD - 2, NCHD):
            b = g & 1
            cps[b].wait()
            pltpu.sync_copy(bufs[b], xs_hbm.at[pl.ds(sbase + g * GCH, GCH)])
        pltpu.sync_copy(wsum_sh.at[pl.ds(sbase, SLOT_W)], wsv_v)
        pltpu.sync_copy(wsv_v, ws_hbm.at[pl.ds(sbase, SLOT_W)])


def _expert_kernel(be_ref, xs_ref, w1_ref, w2_ref, ws_ref, ys_ref):
    x = xs_ref[...]
    mid = jnp.dot(x, w1_ref[0], preferred_element_type=jnp.float32)
    g = mid[:, :FFN]
    u = mid[:, FFN:]
    act = g * jax.lax.logistic(g) * u
    y = jnp.dot(act, w2_ref[0], preferred_element_type=jnp.float32)
    ys_ref[...] = y * ws_ref[0, 0][:, None]


def _combine_kernel(ys_hbm, pos0_hbm, pos1_hbm, out_hbm,
                    idx0a, idx1a, idx0b, idx1b,
                    buf0a, buf1a, buf0b, buf1b, sema, semb):
    cid = lax.axis_index("c")
    sid = lax.axis_index("s")
    wid = sid * NC + cid
    wbase = wid * TOK_W
    idx0s = (idx0a, idx0b)
    idx1s = (idx1a, idx1b)
    buf0s = (buf0a, buf0b)
    buf1s = (buf1a, buf1b)
    sems = (sema, semb)
    cp0s = [None, None]
    cp1s = [None, None]

    def process(b, g):
        cp0s[b].wait()
        cp1s[b].wait()
        b0 = buf0s[b]
        b1 = buf1s[b]

        def row(i, _):
            def col(c, __):
                b0[i, pl.ds(c * L, L)] = (b0[i, pl.ds(c * L, L)]
                                          + b1[i, pl.ds(c * L, L)])
                return 0
            return lax.fori_loop(0, HIDDEN // L, col, 0)

        lax.fori_loop(0, CCH, row, 0)
        pltpu.sync_copy(b0, out_hbm.at[pl.ds(wbase + g * CCH, CCH)])

    for g in range(NCHC):
        b = g & 1
        if g >= 2:
            process(b, g - 2)
        base = wbase + g * CCH
        pltpu.sync_copy(pos0_hbm.at[pl.ds(base, CCH)], idx0s[b])
        pltpu.sync_copy(pos1_hbm.at[pl.ds(base, CCH)], idx1s[b])
        cp0s[b] = pltpu.async_copy(ys_hbm.at[idx0s[b]], buf0s[b], sems[b])
        cp1s[b] = pltpu.async_copy(ys_hbm.at[idx1s[b]], buf1s[b], sems[b])
    for g in range(NCHC - 2, NCHC):
        process(g & 1, g)


def kernel(hidden_states, gate_weight, w1, w2):
    seq, b, h = hidden_states.shape
    x = hidden_states.reshape(T, h)

    pos0, pos1, wt0, wt1, be, laux = pl.pallas_call(
        _gate_kernel,
        out_shape=(
            jax.ShapeDtypeStruct((T,), jnp.int32),
            jax.ShapeDtypeStruct((T,), jnp.int32),
            jax.ShapeDtypeStruct((T,), jnp.float32),
            jax.ShapeDtypeStruct((T,), jnp.float32),
            jax.ShapeDtypeStruct((NBLK,), jnp.int32),
            jax.ShapeDtypeStruct((1, 1), jnp.float32),
        ),
    )(x, gate_weight)

    mesh = plsc.VectorSubcoreMesh(core_axis_name="c", subcore_axis_name="s")
    dispatch = functools.partial(
        pl.kernel,
        out_type=(
            jax.ShapeDtypeStruct((NPAD, HIDDEN), jnp.float32),
            jax.ShapeDtypeStruct((NPAD,), jnp.float32),
        ),
        mesh=mesh,
        scratch_types=[
            pltpu.VMEM((TOK_S,), jnp.int32),
            pltpu.VMEM((TOK_S,), jnp.int32),
            pltpu.VMEM((TOK_S,), jnp.float32),
            pltpu.VMEM((NPAD // NS,), jnp.int32),
            pltpu.VMEM((NPAD // NS,), jnp.float32),
            pltpu.VMEM((GCH,), jnp.int32),
            pltpu.VMEM((GCH,), jnp.int32),
            pltpu.VMEM((GCH, HIDDEN), jnp.float32),
            pltpu.VMEM((GCH, HIDDEN), jnp.float32),
            pltpu.VMEM((SLOT_W,), jnp.float32),
            pltpu.VMEM_SHARED((NPAD,), jnp.int32),
            pltpu.VMEM_SHARED((NPAD,), jnp.float32),
            pltpu.SemaphoreType.DMA,
            pltpu.SemaphoreType.DMA,
        ],
    )(_dispatch_kernel)
    xs, ws = dispatch(x, pos0, pos1, wt0, wt1)

    ys = pl.pallas_call(
        _expert_kernel,
        grid_spec=pltpu.PrefetchScalarGridSpec(
            num_scalar_prefetch=1,
            grid=(NBLK,),
            in_specs=[
                pl.BlockSpec((BLK, HIDDEN), lambda i, be_r: (i, 0)),
                pl.BlockSpec((1, HIDDEN, 2 * FFN), lambda i, be_r: (be_r[i], 0, 0)),
                pl.BlockSpec((1, FFN, HIDDEN), lambda i, be_r: (be_r[i], 0, 0)),
                pl.BlockSpec((1, 1, BLK), lambda i, be_r: (i, 0, 0)),
            ],
            out_specs=pl.BlockSpec((BLK, HIDDEN), lambda i, be_r: (i, 0)),
        ),
        out_shape=jax.ShapeDtypeStruct((NPAD, HIDDEN), jnp.float32),
    )(be, xs, w1, w2, ws.reshape(NBLK, 1, BLK))

    combine = functools.partial(
        pl.kernel,
        out_type=jax.ShapeDtypeStruct((T, HIDDEN), jnp.float32),
        mesh=mesh,
        scratch_types=[
            pltpu.VMEM((CCH,), jnp.int32),
            pltpu.VMEM((CCH,), jnp.int32),
            pltpu.VMEM((CCH,), jnp.int32),
            pltpu.VMEM((CCH,), jnp.int32),
            pltpu.VMEM((CCH, HIDDEN), jnp.float32),
            pltpu.VMEM((CCH, HIDDEN), jnp.float32),
            pltpu.VMEM((CCH, HIDDEN), jnp.float32),
            pltpu.VMEM((CCH, HIDDEN), jnp.float32),
            pltpu.SemaphoreType.DMA,
            pltpu.SemaphoreType.DMA,
        ],
    )(_combine_kernel)
    out = combine(ys, pos0, pos1)

    out = out.reshape(seq, b, h)
    return out, laux[0, 0]


# speedup vs baseline: 1.0351x; 1.0039x over previous
import functools

import jax
import jax.numpy as jnp
from jax import lax
from jax.experimental import pallas as pl
from jax.experimental.pallas import tpu as pltpu
from jax.experimental.pallas import tpu_sc as plsc

E = 8
TOP_K = 2
HIDDEN = 1024
FFN = 1408
ALPHA = 0.001
SCALING = 1.0

BLK = 128
NBLK = 40
NPAD = NBLK * BLK

NC = 2
NS = 16
NW = NC * NS
L = 16

T = 2048
TOK_W = T // NW
TOK_S = T // NS
SLOT_W = NPAD // NW
GCH = 32
NCHD = SLOT_W // GCH
CCH = 16
NCHC = TOK_W // CCH


def _gate_kernel(x_ref, gw_ref, pos0_ref, pos1_ref, wt0_ref, wt1_ref,
                 be_ref, laux_ref):
    x = x_ref[...]
    gw = gw_ref[...]
    logits = jax.lax.dot_general(x, gw, (((1,), (1,)), ((), ())),
                                 preferred_element_type=jnp.float32)
    m = jnp.max(logits, axis=-1, keepdims=True)
    ex = jnp.exp(logits - m)
    s = ex / jnp.sum(ex, axis=-1, keepdims=True)
    iota = jax.lax.broadcasted_iota(jnp.int32, s.shape, 1)
    m1 = jnp.max(s, axis=-1, keepdims=True)
    i1 = jnp.min(jnp.where(s == m1, iota, E), axis=-1, keepdims=True)
    s2 = jnp.where(iota == i1, -1.0, s)
    m2 = jnp.max(s2, axis=-1, keepdims=True)
    i2 = jnp.min(jnp.where(s2 == m2, iota, E), axis=-1, keepdims=True)
    denom = m1 + m2 + 1e-20
    wt0_ref[...] = ((m1 / denom) * SCALING)[:, 0]
    wt1_ref[...] = ((m2 / denom) * SCALING)[:, 0]

    oh0 = (iota == i1).astype(jnp.float32)
    oh1 = (iota == i2).astype(jnp.float32)

    CH = 256
    r = jax.lax.broadcasted_iota(jnp.int32, (CH, CH), 0)
    c = jax.lax.broadcasted_iota(jnp.int32, (CH, CH), 1)
    tril = (r >= c).astype(jnp.float32)
    def chunked_cumsum(oh):
        parts = []
        carry = jnp.zeros((1, E), jnp.float32)
        for j in range(T // CH):
            blk = oh[j * CH:(j + 1) * CH, :]
            loc = jax.lax.dot_general(tril, blk, (((1,), (0,)), ((), ())),
                                      preferred_element_type=jnp.float32)
            parts.append(loc + carry)
            carry = carry + loc[CH - 1:CH, :]
        return jnp.concatenate(parts, axis=0)

    c0 = chunked_cumsum(oh0)
    c1 = chunked_cumsum(oh1)
    counts0 = jnp.sum(oh0, axis=0)
    counts1 = jnp.sum(oh1, axis=0)
    counts = counts0 + counts1
    rank0 = jnp.sum(c0 * oh0, axis=-1) - 1.0
    rank1 = jnp.sum((c1 + counts0[None, :]) * oh1, axis=-1) - 1.0

    counts_i = counts.astype(jnp.int32)
    padded = ((counts_i + (BLK - 1)) // BLK) * BLK
    er = jax.lax.broadcasted_iota(jnp.int32, (E, E), 0)
    ec = jax.lax.broadcasted_iota(jnp.int32, (E, E), 1)
    offs = jnp.sum(jnp.where(ec < er, padded[None, :], 0), axis=1)
    cumblk = jnp.sum(jnp.where(ec <= er, padded[None, :] // BLK, 0), axis=1)

    pos0_ref[...] = (jnp.sum(oh0 * offs[None, :].astype(jnp.float32), axis=-1)
                     + rank0).astype(jnp.int32)
    pos1_ref[...] = (jnp.sum(oh1 * offs[None, :].astype(jnp.float32), axis=-1)
                     + rank1).astype(jnp.int32)

    bi = jax.lax.broadcasted_iota(jnp.int32, (NBLK, E), 0)
    be = jnp.sum((bi >= cumblk[None, :]).astype(jnp.int32), axis=1)
    be_ref[...] = jnp.minimum(be, E - 1)

    ssum = jnp.sum(s, axis=0)
    laux = jnp.sum(ssum * counts) * (ALPHA * E / (T * TOP_K * T))
    laux_ref[...] = laux.reshape(1, 1)


def _dispatch_kernel(x_hbm, pos0_hbm, pos1_hbm, wt0_hbm, wt1_hbm,
                     xs_hbm, ws_hbm,
                     idx_v, val_v, wv_v, zero_v, zerof_v,
                     sidx0_v, sidx1_v, rows0_v, rows1_v,
                     wsv_v, st_sh, wsum_sh, sem0, sem1):
    cid = lax.axis_index("c")
    sid = lax.axis_index("s")
    wid = sid * NC + cid

    with jax.named_scope("disp_zero"):
        z_n = NPAD // NS
        for j in range(z_n // L):
            zero_v[pl.ds(j * L, L)] = jnp.zeros((L,), jnp.int32)
            zerof_v[pl.ds(j * L, L)] = jnp.zeros((L,), jnp.float32)
        pltpu.sync_copy(zero_v, st_sh.at[pl.ds(sid * z_n, z_n)])
        pltpu.sync_copy(zerof_v, wsum_sh.at[pl.ds(sid * z_n, z_n)])
        plsc.subcore_barrier()

    with jax.named_scope("disp_scatter"):
        base = sid * TOK_S
        if True:
            for j in range(TOK_S // L):
                val_v[pl.ds(j * L, L)] = (
                    lax.broadcasted_iota(jnp.int32, (L,), 0) + (base + j * L))
            pltpu.sync_copy(pos0_hbm.at[pl.ds(base, TOK_S)], idx_v)
            pltpu.sync_copy(val_v, st_sh.at[idx_v], add=True)
            pltpu.sync_copy(wt0_hbm.at[pl.ds(base, TOK_S)], wv_v)
            pltpu.sync_copy(wv_v, wsum_sh.at[idx_v], add=True)
            pltpu.sync_copy(pos1_hbm.at[pl.ds(base, TOK_S)], idx_v)
            pltpu.sync_copy(val_v, st_sh.at[idx_v], add=True)
            pltpu.sync_copy(wt1_hbm.at[pl.ds(base, TOK_S)], wv_v)
            pltpu.sync_copy(wv_v, wsum_sh.at[idx_v], add=True)
        plsc.subcore_barrier()

    with jax.named_scope("disp_gather"):
        sbase = wid * SLOT_W
        idxs = (sidx0_v, sidx1_v)
        bufs = (rows0_v, rows1_v)
        sems = (sem0, sem1)
        cps = [None, None]
        for g in range(NCHD):
            b = g & 1
            if g >= 2:
                cps[b].wait()
                pltpu.sync_copy(bufs[b],
                                xs_hbm.at[pl.ds(sbase + (g - 2) * GCH, GCH)])
            pltpu.sync_copy(st_sh.at[pl.ds(sbase + g * GCH, GCH)], idxs[b])
            cps[b] = pltpu.async_copy(x_hbm.at[idxs[b]], bufs[b], sems[b])
        for g in range(NCHD - 2, NCHD):
            b = g & 1
            cps[b].wait()
            pltpu.sync_copy(bufs[b], xs_hbm.at[pl.ds(sbase + g * GCH, GCH)])
        pltpu.sync_copy(wsum_sh.at[pl.ds(sbase, SLOT_W)], wsv_v)
        pltpu.sync_copy(wsv_v, ws_hbm.at[pl.ds(sbase, SLOT_W)])


def _expert_kernel(be_ref, xs_ref, w1_ref, w2_ref, ws_ref, ys_ref):
    x = xs_ref[...]
    mid = jnp.dot(x, w1_ref[0], preferred_element_type=jnp.float32)
    g = mid[:, :FFN]
    u = mid[:, FFN:]
    act = g * jax.lax.logistic(g) * u
    y = jnp.dot(act, w2_ref[0], preferred_element_type=jnp.float32)
    ys_ref[...] = y * ws_ref[0, 0][:, None]


def _combine_kernel(ys_hbm, pos0_hbm, pos1_hbm, out_hbm,
                    idx0a, idx1a, idx0b, idx1b,
                    buf0a, buf1a, buf0b, buf1b, sema, semb):
    cid = lax.axis_index("c")
    sid = lax.axis_index("s")
    wid = sid * NC + cid
    wbase = wid * TOK_W
    idx0s = (idx0a, idx0b)
    idx1s = (idx1a, idx1b)
    buf0s = (buf0a, buf0b)
    buf1s = (buf1a, buf1b)
    sems = (sema, semb)
    cp0s = [None, None]
    cp1s = [None, None]

    def process(b, g):
        cp0s[b].wait()
        cp1s[b].wait()
        b0 = buf0s[b]
        b1 = buf1s[b]

        def row(i, _):
            def col(c, __):
                b0[i, pl.ds(c * L, L)] = (b0[i, pl.ds(c * L, L)]
                                          + b1[i, pl.ds(c * L, L)])
                return 0
            return lax.fori_loop(0, HIDDEN // L, col, 0)

        lax.fori_loop(0, CCH, row, 0)
        pltpu.sync_copy(b0, out_hbm.at[pl.ds(wbase + g * CCH, CCH)])

    for g in range(NCHC):
        b = g & 1
        if g >= 2:
            process(b, g - 2)
        base = wbase + g * CCH
        pltpu.sync_copy(pos0_hbm.at[pl.ds(base, CCH)], idx0s[b])
        pltpu.sync_copy(pos1_hbm.at[pl.ds(base, CCH)], idx1s[b])
        cp0s[b] = pltpu.async_copy(ys_hbm.at[idx0s[b]], buf0s[b], sems[b])
        cp1s[b] = pltpu.async_copy(ys_hbm.at[idx1s[b]], buf1s[b], sems[b])
    for g in range(NCHC - 2, NCHC):
        process(g & 1, g)


def kernel(hidden_states, gate_weight, w1, w2):
    seq, b, h = hidden_states.shape
    x = hidden_states.reshape(T, h)

    pos0, pos1, wt0, wt1, be, laux = pl.pallas_call(
        _gate_kernel,
        out_shape=(
            jax.ShapeDtypeStruct((T,), jnp.int32),
            jax.ShapeDtypeStruct((T,), jnp.int32),
            jax.ShapeDtypeStruct((T,), jnp.float32),
            jax.ShapeDtypeStruct((T,), jnp.float32),
            jax.ShapeDtypeStruct((NBLK,), jnp.int32),
            jax.ShapeDtypeStruct((1, 1), jnp.float32),
        ),
    )(x, gate_weight)

    mesh = plsc.VectorSubcoreMesh(core_axis_name="c", subcore_axis_name="s")
    dispatch = functools.partial(
        pl.kernel,
        out_type=(
            jax.ShapeDtypeStruct((NPAD, HIDDEN), jnp.float32),
            jax.ShapeDtypeStruct((NPAD,), jnp.float32),
        ),
        mesh=mesh,
        scratch_types=[
            pltpu.VMEM((TOK_S,), jnp.int32),
            pltpu.VMEM((TOK_S,), jnp.int32),
            pltpu.VMEM((TOK_S,), jnp.float32),
            pltpu.VMEM((NPAD // NS,), jnp.int32),
            pltpu.VMEM((NPAD // NS,), jnp.float32),
            pltpu.VMEM((GCH,), jnp.int32),
            pltpu.VMEM((GCH,), jnp.int32),
            pltpu.VMEM((GCH, HIDDEN), jnp.float32),
            pltpu.VMEM((GCH, HIDDEN), jnp.float32),
            pltpu.VMEM((SLOT_W,), jnp.float32),
            pltpu.VMEM_SHARED((NPAD,), jnp.int32),
            pltpu.VMEM_SHARED((NPAD,), jnp.float32),
            pltpu.SemaphoreType.DMA,
            pltpu.SemaphoreType.DMA,
        ],
    )(_dispatch_kernel)
    xs, ws = dispatch(x, pos0, pos1, wt0, wt1)

    ys = pl.pallas_call(
        _expert_kernel,
        grid_spec=pltpu.PrefetchScalarGridSpec(
            num_scalar_prefetch=1,
            grid=(NBLK,),
            in_specs=[
                pl.BlockSpec((BLK, HIDDEN), lambda i, be_r: (i, 0)),
                pl.BlockSpec((1, HIDDEN, 2 * FFN), lambda i, be_r: (be_r[i], 0, 0)),
                pl.BlockSpec((1, FFN, HIDDEN), lambda i, be_r: (be_r[i], 0, 0)),
                pl.BlockSpec((1, 1, BLK), lambda i, be_r: (i, 0, 0)),
            ],
            out_specs=pl.BlockSpec((BLK, HIDDEN), lambda i, be_r: (i, 0)),
        ),
        out_shape=jax.ShapeDtypeStruct((NPAD, HIDDEN), jnp.float32),
    )(be, xs, w1, w2, ws.reshape(NBLK, 1, BLK))

    combine = functools.partial(
        pl.kernel,
        out_type=jax.ShapeDtypeStruct((T, HIDDEN), jnp.float32),
        mesh=mesh,
        scratch_types=[
            pltpu.VMEM((CCH,), jnp.int32),
            pltpu.VMEM((CCH,), jnp.int32),
            pltpu.VMEM((CCH,), jnp.int32),
            pltpu.VMEM((CCH,), jnp.int32),
            pltpu.VMEM((CCH, HIDDEN), jnp.float32),
            pltpu.VMEM((CCH, HIDDEN), jnp.float32),
            pltpu.VMEM((CCH, HIDDEN), jnp.float32),
            pltpu.VMEM((CCH, HIDDEN), jnp.float32),
            pltpu.SemaphoreType.DMA,
            pltpu.SemaphoreType.DMA,
        ],
    )(_combine_kernel)
    out = combine(ys, pos0, pos1)

    out = out.reshape(seq, b, h)
    return out, laux[0, 0]


# speedup vs baseline: 1.0813x; 1.0446x over previous
import functools

import jax
import jax.numpy as jnp
from jax import lax
from jax.experimental import pallas as pl
from jax.experimental.pallas import tpu as pltpu
from jax.experimental.pallas import tpu_sc as plsc

E = 8
TOP_K = 2
HIDDEN = 1024
FFN = 1408
ALPHA = 0.001
SCALING = 1.0

BLK = 128
NBLK = 40
NPAD = NBLK * BLK

NC = 2
NS = 16
NW = NC * NS
L = 16

T = 2048
TOK_W = T // NW
TOK_S = T // NS
SLOT_W = NPAD // NW
GCH = 32
NCHD = SLOT_W // GCH
CCH = 16
NCHC = TOK_W // CCH


def _gate_kernel(x_ref, gw_ref, pos0_ref, pos1_ref, wt0_ref, wt1_ref,
                 be_ref, laux_ref):
    x = x_ref[...]
    gw = gw_ref[...]
    logits = jax.lax.dot_general(x, gw, (((1,), (1,)), ((), ())),
                                 preferred_element_type=jnp.float32)
    m = jnp.max(logits, axis=-1, keepdims=True)
    ex = jnp.exp(logits - m)
    s = ex / jnp.sum(ex, axis=-1, keepdims=True)
    iota = jax.lax.broadcasted_iota(jnp.int32, s.shape, 1)
    m1 = jnp.max(s, axis=-1, keepdims=True)
    i1 = jnp.min(jnp.where(s == m1, iota, E), axis=-1, keepdims=True)
    s2 = jnp.where(iota == i1, -1.0, s)
    m2 = jnp.max(s2, axis=-1, keepdims=True)
    i2 = jnp.min(jnp.where(s2 == m2, iota, E), axis=-1, keepdims=True)
    denom = m1 + m2 + 1e-20
    wt0_ref[...] = ((m1 / denom) * SCALING)[:, 0]
    wt1_ref[...] = ((m2 / denom) * SCALING)[:, 0]

    oh0 = (iota == i1).astype(jnp.float32)
    oh1 = (iota == i2).astype(jnp.float32)

    CH = 256
    r = jax.lax.broadcasted_iota(jnp.int32, (CH, CH), 0)
    c = jax.lax.broadcasted_iota(jnp.int32, (CH, CH), 1)
    tril = (r >= c).astype(jnp.float32)
    def chunked_cumsum(oh):
        parts = []
        carry = jnp.zeros((1, E), jnp.float32)
        for j in range(T // CH):
            blk = oh[j * CH:(j + 1) * CH, :]
            loc = jax.lax.dot_general(tril, blk, (((1,), (0,)), ((), ())),
                                      preferred_element_type=jnp.float32)
            parts.append(loc + carry)
            carry = carry + loc[CH - 1:CH, :]
        return jnp.concatenate(parts, axis=0)

    c0 = chunked_cumsum(oh0)
    c1 = chunked_cumsum(oh1)
    counts0 = jnp.sum(oh0, axis=0)
    counts1 = jnp.sum(oh1, axis=0)
    counts = counts0 + counts1
    rank0 = jnp.sum(c0 * oh0, axis=-1) - 1.0
    rank1 = jnp.sum((c1 + counts0[None, :]) * oh1, axis=-1) - 1.0

    counts_i = counts.astype(jnp.int32)
    padded = ((counts_i + (BLK - 1)) // BLK) * BLK
    er = jax.lax.broadcasted_iota(jnp.int32, (E, E), 0)
    ec = jax.lax.broadcasted_iota(jnp.int32, (E, E), 1)
    offs = jnp.sum(jnp.where(ec < er, padded[None, :], 0), axis=1)
    cumblk = jnp.sum(jnp.where(ec <= er, padded[None, :] // BLK, 0), axis=1)

    pos0_ref[...] = (jnp.sum(oh0 * offs[None, :].astype(jnp.float32), axis=-1)
                     + rank0).astype(jnp.int32)
    pos1_ref[...] = (jnp.sum(oh1 * offs[None, :].astype(jnp.float32), axis=-1)
                     + rank1).astype(jnp.int32)

    bi = jax.lax.broadcasted_iota(jnp.int32, (NBLK, E), 0)
    be = jnp.sum((bi >= cumblk[None, :]).astype(jnp.int32), axis=1)
    be_ref[...] = jnp.minimum(be, E - 1)

    ssum = jnp.sum(s, axis=0)
    laux = jnp.sum(ssum * counts) * (ALPHA * E / (T * TOP_K * T))
    laux_ref[...] = laux.reshape(1, 1)


def _dispatch_kernel(x_hbm, pos0_hbm, pos1_hbm, wt0_hbm, wt1_hbm,
                     xs_hbm, ws_hbm,
                     idx_v, val_v, wv_v, zero_v, zerof_v,
                     sidx0_v, sidx1_v, rows0_v, rows1_v,
                     wsv_v, st_sh, wsum_sh, sem0, sem1):
    cid = lax.axis_index("c")
    sid = lax.axis_index("s")
    wid = sid * NC + cid

    with jax.named_scope("disp_zero"):
        z_n = NPAD // NS
        for j in range(z_n // L):
            zero_v[pl.ds(j * L, L)] = jnp.zeros((L,), jnp.int32)
            zerof_v[pl.ds(j * L, L)] = jnp.zeros((L,), jnp.float32)
        pltpu.sync_copy(zero_v, st_sh.at[pl.ds(sid * z_n, z_n)])
        pltpu.sync_copy(zerof_v, wsum_sh.at[pl.ds(sid * z_n, z_n)])
        plsc.subcore_barrier()

    with jax.named_scope("disp_scatter"):
        base = sid * TOK_S
        if True:
            for j in range(TOK_S // L):
                val_v[pl.ds(j * L, L)] = (
                    lax.broadcasted_iota(jnp.int32, (L,), 0) + (base + j * L))
            pltpu.sync_copy(pos0_hbm.at[pl.ds(base, TOK_S)], idx_v)
            pltpu.sync_copy(val_v, st_sh.at[idx_v], add=True)
            pltpu.sync_copy(wt0_hbm.at[pl.ds(base, TOK_S)], wv_v)
            pltpu.sync_copy(wv_v, wsum_sh.at[idx_v], add=True)
            pltpu.sync_copy(pos1_hbm.at[pl.ds(base, TOK_S)], idx_v)
            pltpu.sync_copy(val_v, st_sh.at[idx_v], add=True)
            pltpu.sync_copy(wt1_hbm.at[pl.ds(base, TOK_S)], wv_v)
            pltpu.sync_copy(wv_v, wsum_sh.at[idx_v], add=True)
        plsc.subcore_barrier()

    with jax.named_scope("disp_gather"):
        sbase = wid * SLOT_W
        idxs = (sidx0_v, sidx1_v)
        bufs = (rows0_v, rows1_v)
        sems = (sem0, sem1)
        cps = [None, None]
        for g in range(NCHD):
            b = g & 1
            if g >= 2:
                cps[b].wait()
                pltpu.sync_copy(bufs[b],
                                xs_hbm.at[pl.ds(sbase + (g - 2) * GCH, GCH)])
            pltpu.sync_copy(st_sh.at[pl.ds(sbase + g * GCH, GCH)], idxs[b])
            cps[b] = pltpu.async_copy(x_hbm.at[idxs[b]], bufs[b], sems[b])
        for g in range(NCHD - 2, NCHD):
            b = g & 1
            cps[b].wait()
            pltpu.sync_copy(bufs[b], xs_hbm.at[pl.ds(sbase + g * GCH, GCH)])
        pltpu.sync_copy(wsum_sh.at[pl.ds(sbase, SLOT_W)], wsv_v)
        pltpu.sync_copy(wsv_v, ws_hbm.at[pl.ds(sbase, SLOT_W)])


def _expert_kernel(be_ref, xs_ref, w1_ref, w2_ref, ws_ref, ys_ref):
    x = xs_ref[...]
    mid = jnp.dot(x, w1_ref[0], preferred_element_type=jnp.float32)
    g = mid[:, :FFN]
    u = mid[:, FFN:]
    act = g * jax.lax.logistic(g) * u
    y = jnp.dot(act, w2_ref[0], preferred_element_type=jnp.float32)
    ys_ref[...] = y * ws_ref[0, 0][:, None]


def _combine_kernel(ys_hbm, pos0_hbm, pos1_hbm, out_hbm,
                    idx0a, idx1a, idx0b, idx1b,
                    buf0a, buf1a, buf0b, buf1b, sema, semb):
    cid = lax.axis_index("c")
    sid = lax.axis_index("s")
    wid = sid * NC + cid
    wbase = wid * TOK_W
    idx0s = (idx0a, idx0b)
    idx1s = (idx1a, idx1b)
    buf0s = (buf0a, buf0b)
    buf1s = (buf1a, buf1b)
    sems = (sema, semb)
    cp0s = [None, None]
    cp1s = [None, None]

    def process(b, g):
        cp0s[b].wait()
        cp1s[b].wait()
        b0 = buf0s[b]
        b1 = buf1s[b]

        def row(i, _):
            for c in range(HIDDEN // L):
                b0[i, pl.ds(c * L, L)] = (b0[i, pl.ds(c * L, L)]
                                          + b1[i, pl.ds(c * L, L)])
            return 0

        lax.fori_loop(0, CCH, row, 0)
        pltpu.sync_copy(b0, out_hbm.at[pl.ds(wbase + g * CCH, CCH)])

    for g in range(NCHC):
        b = g & 1
        if g >= 2:
            process(b, g - 2)
        base = wbase + g * CCH
        pltpu.sync_copy(pos0_hbm.at[pl.ds(base, CCH)], idx0s[b])
        pltpu.sync_copy(pos1_hbm.at[pl.ds(base, CCH)], idx1s[b])
        cp0s[b] = pltpu.async_copy(ys_hbm.at[idx0s[b]], buf0s[b], sems[b])
        cp1s[b] = pltpu.async_copy(ys_hbm.at[idx1s[b]], buf1s[b], sems[b])
    for g in range(NCHC - 2, NCHC):
        process(g & 1, g)


def kernel(hidden_states, gate_weight, w1, w2):
    seq, b, h = hidden_states.shape
    x = hidden_states.reshape(T, h)

    pos0, pos1, wt0, wt1, be, laux = pl.pallas_call(
        _gate_kernel,
        out_shape=(
            jax.ShapeDtypeStruct((T,), jnp.int32),
            jax.ShapeDtypeStruct((T,), jnp.int32),
            jax.ShapeDtypeStruct((T,), jnp.float32),
            jax.ShapeDtypeStruct((T,), jnp.float32),
            jax.ShapeDtypeStruct((NBLK,), jnp.int32),
            jax.ShapeDtypeStruct((1, 1), jnp.float32),
        ),
    )(x, gate_weight)

    mesh = plsc.VectorSubcoreMesh(core_axis_name="c", subcore_axis_name="s")
    dispatch = functools.partial(
        pl.kernel,
        out_type=(
            jax.ShapeDtypeStruct((NPAD, HIDDEN), jnp.float32),
            jax.ShapeDtypeStruct((NPAD,), jnp.float32),
        ),
        mesh=mesh,
        scratch_types=[
            pltpu.VMEM((TOK_S,), jnp.int32),
            pltpu.VMEM((TOK_S,), jnp.int32),
            pltpu.VMEM((TOK_S,), jnp.float32),
            pltpu.VMEM((NPAD // NS,), jnp.int32),
            pltpu.VMEM((NPAD // NS,), jnp.float32),
            pltpu.VMEM((GCH,), jnp.int32),
            pltpu.VMEM((GCH,), jnp.int32),
            pltpu.VMEM((GCH, HIDDEN), jnp.float32),
            pltpu.VMEM((GCH, HIDDEN), jnp.float32),
            pltpu.VMEM((SLOT_W,), jnp.float32),
            pltpu.VMEM_SHARED((NPAD,), jnp.int32),
            pltpu.VMEM_SHARED((NPAD,), jnp.float32),
            pltpu.SemaphoreType.DMA,
            pltpu.SemaphoreType.DMA,
        ],
    )(_dispatch_kernel)
    xs, ws = dispatch(x, pos0, pos1, wt0, wt1)

    ys = pl.pallas_call(
        _expert_kernel,
        grid_spec=pltpu.PrefetchScalarGridSpec(
            num_scalar_prefetch=1,
            grid=(NBLK,),
            in_specs=[
                pl.BlockSpec((BLK, HIDDEN), lambda i, be_r: (i, 0)),
                pl.BlockSpec((1, HIDDEN, 2 * FFN), lambda i, be_r: (be_r[i], 0, 0)),
                pl.BlockSpec((1, FFN, HIDDEN), lambda i, be_r: (be_r[i], 0, 0)),
                pl.BlockSpec((1, 1, BLK), lambda i, be_r: (i, 0, 0)),
            ],
            out_specs=pl.BlockSpec((BLK, HIDDEN), lambda i, be_r: (i, 0)),
        ),
        out_shape=jax.ShapeDtypeStruct((NPAD, HIDDEN), jnp.float32),
    )(be, xs, w1, w2, ws.reshape(NBLK, 1, BLK))

    combine = functools.partial(
        pl.kernel,
        out_type=jax.ShapeDtypeStruct((T, HIDDEN), jnp.float32),
        mesh=mesh,
        scratch_types=[
            pltpu.VMEM((CCH,), jnp.int32),
            pltpu.VMEM((CCH,), jnp.int32),
            pltpu.VMEM((CCH,), jnp.int32),
            pltpu.VMEM((CCH,), jnp.int32),
            pltpu.VMEM((CCH, HIDDEN), jnp.float32),
            pltpu.VMEM((CCH, HIDDEN), jnp.float32),
            pltpu.VMEM((CCH, HIDDEN), jnp.float32),
            pltpu.VMEM((CCH, HIDDEN), jnp.float32),
            pltpu.SemaphoreType.DMA,
            pltpu.SemaphoreType.DMA,
        ],
    )(_combine_kernel)
    out = combine(ys, pos0, pos1)

    out = out.reshape(seq, b, h)
    return out, laux[0, 0]
